# baseline (speedup 1.0000x reference)
import functools
import jax
import jax.numpy as jnp
from jax import lax
from jax.experimental import pallas as pl
from jax.experimental.pallas import tpu as pltpu
from jax.experimental.pallas import tpu_sc as plsc

N = 10000
NP = 10240
D = 128
NC = 2
NS = 16
NW = NC * NS
CBP = 128
DEGW = 16
RPT = NP // NS


def _mesh():
    return plsc.VectorSubcoreMesh(core_axis_name="c", subcore_axis_name="s")


def _deg_partials(dst16, zeros80, iota80):
    EPW = dst16.shape[1]

    @functools.partial(
        pl.kernel,
        mesh=_mesh(),
        out_type=jax.ShapeDtypeStruct((NC, NP // D, D), jnp.float32),
        scratch_types=[
            pltpu.VMEM((EPW, 16), jnp.int32),
            pltpu.VMEM((NP // D, D), jnp.float32),
            pltpu.VMEM((1, NP // D), jnp.int32),
            pltpu.VMEM_SHARED((NP // D, D), jnp.float32),
        ],
        compiler_params=pltpu.CompilerParams(needs_layout_passes=False),
    )
    def k(dst_hbm, z_hbm, id_hbm, out_hbm, dst_v, hist, id_v, acc):
        cid = lax.axis_index("c")
        sid = lax.axis_index("s")
        wid = sid * NC + cid
        pltpu.sync_copy(z_hbm, hist)
        pltpu.sync_copy(id_hbm, id_v)
        pltpu.sync_copy(dst_hbm.at[wid], dst_v)

        @pl.when(sid == 0)
        def _zero_acc():
            pltpu.sync_copy(z_hbm, acc)

        ones = jnp.ones((16,), jnp.float32)

        def body(j, carry):
            idx = dst_v[j]
            plsc.addupdate_scatter(hist, [idx >> 7, idx & 127], ones)
            return carry

        lax.fori_loop(0, EPW, body, 0)
        plsc.subcore_barrier()
        pltpu.sync_copy(hist, acc.at[id_v.at[0]], add=True)
        plsc.subcore_barrier()

        @pl.when(sid < 10)
        def _writeout():
            stripe = pl.ds(sid * 8, 8)
            pltpu.sync_copy(acc.at[stripe], out_hbm.at[cid, stripe])

    return k(dst16, zeros80, iota80)


CB = 80


def _agg_partials(h, src3, dst3, z_rows):
    CH = src3.shape[1]

    @functools.partial(
        pl.kernel,
        mesh=_mesh(),
        out_type=jax.ShapeDtypeStruct((NC, NP, D), jnp.float32),
        scratch_types=[
            pltpu.VMEM((CH, CB), jnp.int32),
            pltpu.VMEM((CH, CB), jnp.int32),
            pltpu.VMEM((CB, D), jnp.float32),
            pltpu.VMEM_SHARED((NP, D), jnp.float32),
        ],
    )
    def k(h_hbm, src_hbm, dst_hbm, z_hbm, out_hbm,
          src_v, dst_v, ra, acc):
        cid = lax.axis_index("c")
        sid = lax.axis_index("s")
        wid = sid * NC + cid
        stripe = pl.ds(sid * RPT, RPT)
        pltpu.sync_copy(z_hbm, acc.at[stripe])
        pltpu.sync_copy(src_hbm.at[wid], src_v)
        pltpu.sync_copy(dst_hbm.at[wid], dst_v)
        plsc.subcore_barrier()

        def body(j, carry):
            pltpu.sync_copy(h_hbm.at[src_v.at[j]], ra)
            pltpu.sync_copy(ra, acc.at[dst_v.at[j]], add=True)
            return carry

        lax.fori_loop(0, CH, body, 0)
        plsc.subcore_barrier()
        pltpu.sync_copy(acc.at[stripe], out_hbm.at[cid, stripe])

    return k(h, src3, dst3, z_rows)


R = 400


def _tc_pre(x, W1, deg_col):
    def body(x_ref, w_ref, d_ref, hp_ref, dinv_ref):
        dinv = jnp.broadcast_to(lax.rsqrt(1.0 + d_ref[...]), (R, D))
        h = jnp.dot(x_ref[...], w_ref[...], preferred_element_type=jnp.float32)
        hp_ref[...] = h * dinv
        dinv_ref[...] = dinv

    return pl.pallas_call(
        body,
        grid=(N // R,),
        in_specs=[
            pl.BlockSpec((R, D), lambda i: (i, 0)),
            pl.BlockSpec((D, D), lambda i: (0, 0)),
            pl.BlockSpec((R, 1), lambda i: (i, 0)),
        ],
        out_specs=[pl.BlockSpec((R, D), lambda i: (i, 0))] * 2,
        out_shape=[jax.ShapeDtypeStruct((N, D), jnp.float32)] * 2,
    )(x, W1, deg_col)


def _tc_mid(aggp, hp, dinv, b1, W2):
    def body(a0, a1, hpr, dv, b, w, out):
        z = dv[...] * (a0[...] + a1[...] + hpr[...]) + b[...]
        r = jnp.maximum(z, 0.0)
        out[...] = jnp.dot(r, w[...], preferred_element_type=jnp.float32) * dv[...]

    return pl.pallas_call(
        body,
        grid=(N // R,),
        in_specs=[
            pl.BlockSpec((R, D), lambda i: (i, 0)),
            pl.BlockSpec((R, D), lambda i: (i, 0)),
            pl.BlockSpec((R, D), lambda i: (i, 0)),
            pl.BlockSpec((R, D), lambda i: (i, 0)),
            pl.BlockSpec((1, D), lambda i: (0, 0)),
            pl.BlockSpec((D, D), lambda i: (0, 0)),
        ],
        out_specs=pl.BlockSpec((R, D), lambda i: (i, 0)),
        out_shape=jax.ShapeDtypeStruct((N, D), jnp.float32),
    )(aggp[0], aggp[1], hp, dinv, b1, W2)


def _tc_fin(aggp, hp, dinv, b2):
    def body(a0, a1, hpr, dv, b, out):
        z = dv[...] * (a0[...] + a1[...] + hpr[...]) + b[...]
        m = jnp.max(z, axis=1, keepdims=True)
        e = jnp.exp(z - m)
        s = jnp.sum(e, axis=1, keepdims=True)
        out[...] = (z - m) - jnp.log(s)

    return pl.pallas_call(
        body,
        grid=(N // R,),
        in_specs=[
            pl.BlockSpec((R, D), lambda i: (i, 0)),
            pl.BlockSpec((R, D), lambda i: (i, 0)),
            pl.BlockSpec((R, D), lambda i: (i, 0)),
            pl.BlockSpec((R, D), lambda i: (i, 0)),
            pl.BlockSpec((1, D), lambda i: (0, 0)),
        ],
        out_specs=pl.BlockSpec((R, D), lambda i: (i, 0)),
        out_shape=jax.ShapeDtypeStruct((N, D), jnp.float32),
    )(aggp[0], aggp[1], hp, dinv, b2)


def kernel(x, edge_index, W1, b1, W2, b2):
    E = edge_index.shape[1]
    per_w = E // NW
    assert per_w * NW == E
    CH = per_w // CB
    assert CH * CB == per_w

    src3 = edge_index[0].reshape(NW, CH, CB)
    dst3 = edge_index[1].reshape(NW, CH, CB)

    dst16 = edge_index[1].reshape(NW, per_w // 16, 16)
    zeros80 = jnp.zeros((NP // D, D), jnp.float32)
    iota80 = jnp.arange(NP // D, dtype=jnp.int32).reshape(1, NP // D)
    zeros_rows = jnp.zeros((RPT, D), jnp.float32)

    degp = _deg_partials(dst16, zeros80, iota80)
    deg_col = (degp[0] + degp[1]).reshape(NP, 1)[:N]
    hp1, dinv = _tc_pre(x, W1, deg_col)
    agg1 = _agg_partials(hp1, src3, dst3, zeros_rows)
    hp2 = _tc_mid(agg1, hp1, dinv, b1.reshape(1, D), W2)
    agg2 = _agg_partials(hp2, src3, dst3, zeros_rows)
    return _tc_fin(agg2, hp2, dinv, b2.reshape(1, D))


# speedup vs baseline: 1.0010x; 1.0010x over previous
import functools
import jax
import jax.numpy as jnp
from jax import lax
from jax.experimental import pallas as pl
from jax.experimental.pallas import tpu as pltpu
from jax.experimental.pallas import tpu_sc as plsc

N = 10000
NP = 10240
D = 128
NC = 2
NS = 16
NW = NC * NS
RPT = NP // NS


def _mesh():
    return plsc.VectorSubcoreMesh(core_axis_name="c", subcore_axis_name="s")


def _deg_partials(dst16, zeros80, iota80):
    EPW = dst16.shape[1]

    @functools.partial(
        pl.kernel,
        mesh=_mesh(),
        out_type=jax.ShapeDtypeStruct((NC, NP // D, D), jnp.float32),
        scratch_types=[
            pltpu.VMEM((EPW, 16), jnp.int32),
            pltpu.VMEM((NP // D, D), jnp.float32),
            pltpu.VMEM((1, NP // D), jnp.int32),
            pltpu.VMEM_SHARED((NP // D, D), jnp.float32),
        ],
        compiler_params=pltpu.CompilerParams(needs_layout_passes=False),
    )
    def k(dst_hbm, z_hbm, id_hbm, out_hbm, dst_v, hist, id_v, acc):
        cid = lax.axis_index("c")
        sid = lax.axis_index("s")
        wid = sid * NC + cid
        pltpu.sync_copy(z_hbm, hist)
        pltpu.sync_copy(id_hbm, id_v)
        pltpu.sync_copy(dst_hbm.at[wid], dst_v)

        @pl.when(sid == 0)
        def _zero_acc():
            pltpu.sync_copy(z_hbm, acc)

        ones = jnp.ones((16,), jnp.float32)

        def body(j, carry):
            idx = dst_v[j]
            plsc.addupdate_scatter(hist, [idx >> 7, idx & 127], ones)
            return carry

        lax.fori_loop(0, EPW, body, 0)
        plsc.subcore_barrier()
        pltpu.sync_copy(hist, acc.at[id_v.at[0]], add=True)
        plsc.subcore_barrier()

        @pl.when(sid < 10)
        def _writeout():
            stripe = pl.ds(sid * 8, 8)
            pltpu.sync_copy(acc.at[stripe], out_hbm.at[cid, stripe])

    return k(dst16, zeros80, iota80)


CB = 80


def _agg_partials(h, src3, dst3, z_rows):
    CH = src3.shape[1]

    @functools.partial(
        pl.kernel,
        mesh=_mesh(),
        out_type=jax.ShapeDtypeStruct((NC, NP, D), jnp.float32),
        scratch_types=[
            pltpu.VMEM((CH, CB), jnp.int32),
            pltpu.VMEM((CH, CB), jnp.int32),
            pltpu.VMEM((CB, D), jnp.float32),
            pltpu.VMEM_SHARED((NP, D), jnp.float32),
        ],
    )
    def k(h_hbm, src_hbm, dst_hbm, z_hbm, out_hbm,
          src_v, dst_v, ra, acc):
        cid = lax.axis_index("c")
        sid = lax.axis_index("s")
        wid = sid * NC + cid
        stripe = pl.ds(sid * RPT, RPT)
        pltpu.sync_copy(z_hbm, acc.at[stripe])
        pltpu.sync_copy(src_hbm.at[wid], src_v)
        pltpu.sync_copy(dst_hbm.at[wid], dst_v)
        plsc.subcore_barrier()

        def body(j, carry):
            pltpu.sync_copy(h_hbm.at[src_v.at[j]], ra)
            pltpu.sync_copy(ra, acc.at[dst_v.at[j]], add=True)
            return carry

        lax.fori_loop(0, CH, body, 0)
        plsc.subcore_barrier()
        pltpu.sync_copy(acc.at[stripe], out_hbm.at[cid, stripe])

    return k(h, src3, dst3, z_rows)


R = 400


def _tc_pre(x, W1, deg_col):
    def body(x_ref, w_ref, d_ref, hp_ref, dinv_ref):
        dinv = jnp.broadcast_to(lax.rsqrt(1.0 + d_ref[...]), (R, D))
        h = jnp.dot(x_ref[...], w_ref[...], preferred_element_type=jnp.float32)
        hp_ref[...] = h * dinv
        dinv_ref[...] = dinv

    return pl.pallas_call(
        body,
        grid=(N // R,),
        in_specs=[
            pl.BlockSpec((R, D), lambda i: (i, 0)),
            pl.BlockSpec((D, D), lambda i: (0, 0)),
            pl.BlockSpec((R, 1), lambda i: (i, 0)),
        ],
        out_specs=[pl.BlockSpec((R, D), lambda i: (i, 0))] * 2,
        out_shape=[jax.ShapeDtypeStruct((N, D), jnp.float32)] * 2,
    )(x, W1, deg_col)


def _tc_mid(aggp, hp, dinv, b1, W2):
    def body(a0, a1, hpr, dv, b, w, out):
        z = dv[...] * (a0[...] + a1[...] + hpr[...]) + b[...]
        r = jnp.maximum(z, 0.0)
        out[...] = jnp.dot(r, w[...], preferred_element_type=jnp.float32) * dv[...]

    return pl.pallas_call(
        body,
        grid=(N // R,),
        in_specs=[
            pl.BlockSpec((R, D), lambda i: (i, 0)),
            pl.BlockSpec((R, D), lambda i: (i, 0)),
            pl.BlockSpec((R, D), lambda i: (i, 0)),
            pl.BlockSpec((R, D), lambda i: (i, 0)),
            pl.BlockSpec((1, D), lambda i: (0, 0)),
            pl.BlockSpec((D, D), lambda i: (0, 0)),
        ],
        out_specs=pl.BlockSpec((R, D), lambda i: (i, 0)),
        out_shape=jax.ShapeDtypeStruct((N, D), jnp.float32),
    )(aggp[0], aggp[1], hp, dinv, b1, W2)


def _tc_fin(aggp, hp, dinv, b2):
    def body(a0, a1, hpr, dv, b, out):
        z = dv[...] * (a0[...] + a1[...] + hpr[...]) + b[...]
        m = jnp.max(z, axis=1, keepdims=True)
        e = jnp.exp(z - m)
        s = jnp.sum(e, axis=1, keepdims=True)
        out[...] = (z - m) - jnp.log(s)

    return pl.pallas_call(
        body,
        grid=(N // R,),
        in_specs=[
            pl.BlockSpec((R, D), lambda i: (i, 0)),
            pl.BlockSpec((R, D), lambda i: (i, 0)),
            pl.BlockSpec((R, D), lambda i: (i, 0)),
            pl.BlockSpec((R, D), lambda i: (i, 0)),
            pl.BlockSpec((1, D), lambda i: (0, 0)),
        ],
        out_specs=pl.BlockSpec((R, D), lambda i: (i, 0)),
        out_shape=jax.ShapeDtypeStruct((N, D), jnp.float32),
    )(aggp[0], aggp[1], hp, dinv, b2)


def kernel(x, edge_index, W1, b1, W2, b2):
    E = edge_index.shape[1]
    per_w = E // NW
    assert per_w * NW == E
    CH = per_w // CB
    assert CH * CB == per_w

    src3 = edge_index[0].reshape(NW, CH, CB)
    dst3 = edge_index[1].reshape(NW, CH, CB)

    dst16 = edge_index[1].reshape(NW, per_w // 16, 16)
    zeros80 = jnp.zeros((NP // D, D), jnp.float32)
    iota80 = jnp.arange(NP // D, dtype=jnp.int32).reshape(1, NP // D)
    zeros_rows = jnp.zeros((RPT, D), jnp.float32)

    degp = _deg_partials(dst16, zeros80, iota80)
    deg_col = (degp[0] + degp[1]).reshape(NP, 1)[:N]
    hp1, dinv = _tc_pre(x, W1, deg_col)
    agg1 = _agg_partials(hp1, src3, dst3, zeros_rows)
    hp2 = _tc_mid(agg1, hp1, dinv, b1.reshape(1, D), W2)
    agg2 = _agg_partials(hp2, src3, dst3, zeros_rows)
    return _tc_fin(agg2, hp2, dinv, b2.reshape(1, D))
